# baseline (speedup 1.0000x reference)
import jax
import jax.numpy as jnp
from jax.experimental import pallas as pl
from jax.experimental.pallas import tpu as pltpu


_LX = 4


def _add_body(x_ref, t_ref, o_ref):
    half = pl.program_id(0) % 2
    tsl = t_ref[pl.ds(_LX * half, _LX), :]
    o_ref[...] = x_ref[...] + jax.lax.broadcast_in_dim(
        tsl, o_ref.shape, (0, 1)
    )


def kernel(x, pos_table):
    B, L, D = x.shape
    xt = x.transpose(1, 2, 0)
    out_t = pl.pallas_call(
        _add_body,
        grid=(L // _LX,),
        in_specs=[
            pl.BlockSpec((_LX, D, B), lambda i: (i, 0, 0)),
            pl.BlockSpec((8, D), lambda i: (i // 2, 0)),
        ],
        out_specs=pl.BlockSpec((_LX, D, B), lambda i: (i, 0, 0)),
        out_shape=jax.ShapeDtypeStruct((L, D, B), x.dtype),
        compiler_params=pltpu.CompilerParams(
            dimension_semantics=("arbitrary",),
        ),
    )(xt, pos_table)
    return out_t.transpose(2, 0, 1)
